# baseline (speedup 1.0000x reference)
import functools

import jax
import jax.numpy as jnp
from jax import lax
from jax.experimental import pallas as pl
from jax.experimental.pallas import tpu as pltpu
from jax.experimental.pallas import tpu_sc as plsc

B = 16384
F = 26
V = 1000000
D = 13
NC = 2
NS = 16
NW = NC * NS
NB = B // NW
L = 16
CHUNK = 128
GR = (F * NB) // CHUNK
RPF = NB // CHUNK

_mesh = plsc.VectorSubcoreMesh(core_axis_name="c", subcore_axis_name="s")


@functools.partial(
    pl.kernel,
    out_type=jax.ShapeDtypeStruct((B,), jnp.float32),
    mesh=_mesh,
    scratch_types=[
        pltpu.VMEM((GR, CHUNK), jnp.int32),
        pltpu.VMEM((F * L,), jnp.int32),
        pltpu.VMEM((GR * CHUNK,), jnp.float32),
        pltpu.VMEM((D, NB), jnp.float32),
        pltpu.VMEM((64,), jnp.float32),
        pltpu.VMEM((NB,), jnp.float32),
        pltpu.SemaphoreType.DMA,
        pltpu.SemaphoreType.DMA,
    ],
    compiler_params=pltpu.CompilerParams(needs_layout_passes=False),
)
def _logreg_sc(sparse_hbm, dense_hbm, tables_hbm, wb_hbm, ring_hbm, out_hbm,
               idx_v, ring_v, val_v, den_v, wb_s, acc_v, sem, sem2):
    wid = lax.axis_index("s") * NC + lax.axis_index("c")
    base = wid * NB

    with jax.named_scope("stage_in"):
        den_cp = pltpu.async_copy(dense_hbm.at[wid], den_v, sem2)
        wb_cp = pltpu.async_copy(wb_hbm, wb_s, sem2)
        pltpu.sync_copy(ring_hbm, ring_v)
        pltpu.sync_copy(sparse_hbm.at[wid], idx_v)

    def off_fire(g, carry):
        for c in range(CHUNK // L):
            sl = pl.ds(c * L, L)
            rpos = lax.rem(g * (CHUNK // L) + c, F // 2) * L
            idx_v[g, sl] = idx_v[g, sl] + ring_v[pl.ds(rpos, L)]
        pltpu.async_copy(tables_hbm.at[idx_v.at[g]],
                         val_v.at[pl.ds(g * CHUNK, CHUNK)], sem)
        return carry

    with jax.named_scope("off_fire"):
        lax.fori_loop(0, GR, off_fire, 0)

    with jax.named_scope("stage_wait"):
        den_cp.wait()
        wb_cp.wait()

    wv0 = wb_s[pl.ds(0, L)]
    wv1 = wb_s[pl.ds(L, L)]
    wv2 = wb_s[pl.ds(2 * L, L)]
    wvs = (wv0, wv1, wv2)

    def wsc(k):
        return wvs[k // L][k % L]

    def dense_init(cidx, carry):
        acc = jnp.full((L,), wsc(F + D), jnp.float32)
        for dd in range(D):
            acc = acc + den_v[dd, pl.ds(cidx * L, L)] * wsc(F + dd)
        acc_v[pl.ds(cidx * L, L)] = acc
        return carry

    with jax.named_scope("dense_init"):
        lax.fori_loop(0, NB // L, dense_init, 0)

    with jax.named_scope("drain"):
        pltpu.make_async_copy(
            tables_hbm.at[pl.ds(0, GR * CHUNK)], val_v, sem).wait()

    lanes_f = lax.iota(jnp.int32, L) * F

    def accum(cidx, carry):
        sl = pl.ds(cidx * L, L)
        acc = acc_v[sl]
        cbase = cidx * L * F
        for f in range(F):
            ev = plsc.load_gather(val_v, [lanes_f + (cbase + f)])
            acc = acc + ev * wsc(f)
        acc_v[sl] = acc
        return carry

    with jax.named_scope("accum"):
        lax.fori_loop(0, NB // L, accum, 0)

    with jax.named_scope("write_out"):
        pltpu.sync_copy(acc_v, out_hbm.at[pl.ds(base, NB)])


def kernel(sparse, dense, tables, W, b):
    sparse_r = sparse.reshape(NW, GR, CHUNK)
    dense_t = dense.reshape(NW, NB, D).transpose(0, 2, 1)
    wb = jnp.concatenate([W[:, 0], b, jnp.zeros((64 - F - D - 1,), jnp.float32)])
    ring = (jnp.tile(jnp.arange(F, dtype=jnp.int32), 2 * L)[: F * L] * V)
    out = _logreg_sc(sparse_r, dense_t, tables, wb, ring)
    return out[:, None]


# speedup vs baseline: 1.2697x; 1.2697x over previous
import functools

import jax
import jax.numpy as jnp
from jax import lax
from jax.experimental import pallas as pl
from jax.experimental.pallas import tpu as pltpu
from jax.experimental.pallas import tpu_sc as plsc

B = 16384
F = 26
V = 1000000
D = 13
NC = 2
NS = 16
NW = NC * NS
NB = B // NW
L = 16
CHUNK = 128
GR = (F * NB) // CHUNK
RPF = NB // CHUNK

_mesh = plsc.VectorSubcoreMesh(core_axis_name="c", subcore_axis_name="s")


@functools.partial(
    pl.kernel,
    out_type=jax.ShapeDtypeStruct((B,), jnp.float32),
    mesh=_mesh,
    scratch_types=[
        pltpu.VMEM((GR, CHUNK), jnp.int32),
        pltpu.VMEM((GR * CHUNK,), jnp.float32),
        pltpu.VMEM((D, NB), jnp.float32),
        pltpu.VMEM((64,), jnp.float32),
        pltpu.VMEM((NB,), jnp.float32),
        pltpu.SemaphoreType.DMA,
        pltpu.SemaphoreType.DMA,
    ],
    compiler_params=pltpu.CompilerParams(needs_layout_passes=False),
)
def _logreg_sc(sparse_hbm, dense_hbm, tables_hbm, wb_hbm, out_hbm,
               idx_v, val_v, den_v, wb_s, acc_v, sem, sem2):
    wid = lax.axis_index("s") * NC + lax.axis_index("c")
    base = wid * NB

    den_cp = pltpu.async_copy(dense_hbm.at[wid], den_v, sem2)
    wb_cp = pltpu.async_copy(wb_hbm, wb_s, sem2)
    pltpu.sync_copy(sparse_hbm.at[wid], idx_v)

    def off_fire(g, carry):
        off = (g // RPF) * V
        for c in range(CHUNK // L):
            sl = pl.ds(c * L, L)
            idx_v[g, sl] = idx_v[g, sl] + off
        pltpu.async_copy(tables_hbm.at[idx_v.at[g]],
                         val_v.at[pl.ds(g * CHUNK, CHUNK)], sem)
        return carry

    lax.fori_loop(0, GR, off_fire, 0)

    den_cp.wait()
    wb_cp.wait()

    wv0 = wb_s[pl.ds(0, L)]
    wv1 = wb_s[pl.ds(L, L)]
    wv2 = wb_s[pl.ds(2 * L, L)]
    wvs = (wv0, wv1, wv2)

    def wsc(k):
        return wvs[k // L][k % L]

    def dense_init(cidx, carry):
        acc = jnp.full((L,), wsc(F + D), jnp.float32)
        for dd in range(D):
            acc = acc + den_v[dd, pl.ds(cidx * L, L)] * wsc(F + dd)
        acc_v[pl.ds(cidx * L, L)] = acc
        return carry

    lax.fori_loop(0, NB // L, dense_init, 0)

    pltpu.make_async_copy(
        tables_hbm.at[pl.ds(0, GR * CHUNK)], val_v, sem).wait()

    def accum(cidx, carry):
        row_in_f = cidx // (CHUNK // L)
        lane_off = (cidx % (CHUNK // L)) * L
        sl = pl.ds(cidx * L, L)
        acc = acc_v[sl]
        for f in range(F):
            acc = acc + val_v[pl.ds((f * RPF + row_in_f) * CHUNK + lane_off, L)] * wsc(f)
        acc_v[sl] = acc
        return carry

    lax.fori_loop(0, NB // L, accum, 0)

    pltpu.sync_copy(acc_v, out_hbm.at[pl.ds(base, NB)])


def kernel(sparse, dense, tables, W, b):
    sparse_t = sparse.reshape(NW, NB, F).transpose(0, 2, 1).reshape(NW, GR, CHUNK)
    dense_t = dense.reshape(NW, NB, D).transpose(0, 2, 1)
    wb = jnp.concatenate([W[:, 0], b, jnp.zeros((64 - F - D - 1,), jnp.float32)])
    out = _logreg_sc(sparse_t, dense_t, tables, wb)
    return out[:, None]
